# baseline (speedup 1.0000x reference)
import functools

import jax
import jax.numpy as jnp
from jax import lax
from jax.experimental import pallas as pl
from jax.experimental.pallas import tpu as pltpu
from jax.experimental.pallas import tpu_sc as plsc

NC = 2
NS = 16
NW = NC * NS
LANE = 16


def _mesh():
    return plsc.VectorSubcoreMesh(core_axis_name="c", subcore_axis_name="s")


def _fill_1d(ref, n, value):
    def body(i, _):
        ref[pl.ds(i * LANE, LANE)] = jnp.full((LANE,), value, jnp.float32)
        return 0
    lax.fori_loop(0, n // LANE, body, 0)


def _fill_2d(ref, rows, cols, value):
    def rbody(r, _):
        def cbody(k, _2):
            ref[r, pl.ds(k * LANE, LANE)] = jnp.full((LANE,), value, jnp.float32)
            return 0
        lax.fori_loop(0, cols // LANE, cbody, 0)
        return 0
    lax.fori_loop(0, rows, rbody, 0)


def _sc_degree(dst, n_pad):
    e_total = dst.shape[0]
    epw = e_total // NW
    C = 2000
    nchunks = epw // C
    rpt = n_pad // NS

    @functools.partial(
        pl.kernel,
        out_type=jax.ShapeDtypeStruct((NC, n_pad), jnp.float32),
        mesh=_mesh(),
        scratch_types=[
            pltpu.VMEM((C,), jnp.int32),
            pltpu.VMEM((C,), jnp.float32),
            pltpu.VMEM((rpt,), jnp.float32),
            pltpu.VMEM_SHARED((n_pad,), jnp.float32),
        ],
    )
    def deg_kernel(dst_hbm, out_hbm, idx_v, ones_v, zbuf_v, deg_sh):
        c = lax.axis_index("c")
        s = lax.axis_index("s")
        wid = s * NC + c
        _fill_1d(ones_v, C, 1.0)
        _fill_1d(zbuf_v, rpt, 0.0)
        pltpu.sync_copy(zbuf_v, deg_sh.at[pl.ds(s * rpt, rpt)])
        plsc.subcore_barrier()

        def body(j, _):
            pltpu.sync_copy(dst_hbm.at[pl.ds(wid * epw + j * C, C)], idx_v)
            pltpu.sync_copy(ones_v, deg_sh.at[idx_v], add=True)
            return 0
        lax.fori_loop(0, nchunks, body, 0)
        plsc.subcore_barrier()
        pltpu.sync_copy(deg_sh.at[pl.ds(s * rpt, rpt)],
                        out_hbm.at[c, pl.ds(s * rpt, rpt)])

    return deg_kernel(dst)


def _tc_encode(x, w, deg2):
    n, d = x.shape
    h_dim = w.shape[1]
    blk = 512

    n_pad = deg2.shape[1]

    def body(x_ref, w_ref, deg_ref, g_ref):
        i = pl.program_id(0)
        deg = (deg_ref[0, pl.ds(i * blk, blk)]
               + deg_ref[1, pl.ds(i * blk, blk)] + 1.0)
        dinv = lax.rsqrt(deg)
        h = jnp.dot(x_ref[...], w_ref[...], preferred_element_type=jnp.float32)
        g_ref[...] = h * dinv[:, None]

    return pl.pallas_call(
        body,
        grid=((n + blk - 1) // blk,),
        in_specs=[
            pl.BlockSpec((blk, d), lambda i: (i, 0)),
            pl.BlockSpec((d, h_dim), lambda i: (0, 0)),
            pl.BlockSpec((2, n_pad), lambda i: (0, 0)),
        ],
        out_specs=pl.BlockSpec((blk, h_dim), lambda i: (i, 0)),
        out_shape=jax.ShapeDtypeStruct((n, h_dim), jnp.float32),
    )(x, w, deg2)


def _sc_scatter(g, src, dst, n_pad):
    n, h_dim = g.shape
    e_total = src.shape[0]
    epw = e_total // NW
    C = 80
    nchunks = epw // C
    rpt = n_pad // NS
    zrows = 64

    @functools.partial(
        pl.kernel,
        out_type=jax.ShapeDtypeStruct((NC, n_pad, h_dim), jnp.float32),
        mesh=_mesh(),
        scratch_types=[
            pltpu.VMEM((C,), jnp.int32),
            pltpu.VMEM((C,), jnp.int32),
            pltpu.VMEM((C,), jnp.int32),
            pltpu.VMEM((C,), jnp.int32),
            pltpu.VMEM((C, h_dim), jnp.float32),
            pltpu.VMEM((C, h_dim), jnp.float32),
            pltpu.VMEM((zrows, h_dim), jnp.float32),
            pltpu.VMEM_SHARED((n_pad, h_dim), jnp.float32),
            pltpu.SemaphoreType.DMA,
            pltpu.SemaphoreType.DMA,
        ],
    )
    def scat_kernel(g_hbm, src_hbm, dst_hbm, out_hbm,
                    sidx0_v, sidx1_v, didx0_v, didx1_v, rows0_v, rows1_v,
                    zbuf_v, acc_sh, sem0, sem1):
        c = lax.axis_index("c")
        s = lax.axis_index("s")
        wid = s * NC + c
        ebase = wid * epw
        sems = (sem0, sem1)
        sidxs = (sidx0_v, sidx1_v)
        didxs = (didx0_v, didx1_v)
        rows = (rows0_v, rows1_v)
        _fill_2d(zbuf_v, zrows, h_dim, 0.0)

        def zinit(k, _):
            pltpu.sync_copy(zbuf_v, acc_sh.at[pl.ds(s * rpt + k * zrows, zrows)])
            return 0
        lax.fori_loop(0, rpt // zrows, zinit, 0)
        plsc.subcore_barrier()

        def fetch(j, b):
            base = ebase + j * C
            pltpu.sync_copy(src_hbm.at[pl.ds(base, C)], sidxs[b])
            pltpu.sync_copy(dst_hbm.at[pl.ds(base, C)], didxs[b])
            pltpu.async_copy(g_hbm.at[sidxs[b]], rows[b], sems[b])

        def drain(b):
            pltpu.make_async_copy(g_hbm.at[sidxs[b]], rows[b],
                                  sems[b]).wait()

        def process(b):
            pltpu.sync_copy(rows[b], acc_sh.at[didxs[b]], add=True)

        fetch(0, 0)

        def body(gi, _):
            j0 = gi * 2
            fetch(j0 + 1, 1)
            drain(0)
            process(0)
            fetch(lax.rem(j0 + 2, nchunks), 0)
            drain(1)
            process(1)
            return 0
        lax.fori_loop(0, nchunks // 2, body, 0)
        drain(0)
        if nchunks % 2 == 1:
            process(0)
        plsc.subcore_barrier()

        def wback(k, _):
            off = s * rpt + k * zrows
            pltpu.sync_copy(acc_sh.at[pl.ds(off, zrows)],
                            out_hbm.at[c, pl.ds(off, zrows)])
            return 0
        lax.fori_loop(0, rpt // zrows, wback, 0)

    return scat_kernel(g, src, dst)


def _tc_normalize(acc2, g, deg2, b2):
    n, h_dim = g.shape
    blk = 512

    n_pad = deg2.shape[1]

    def body(a_ref, g_ref, deg_ref, b_ref, z_ref):
        i = pl.program_id(0)
        deg = (deg_ref[0, pl.ds(i * blk, blk)]
               + deg_ref[1, pl.ds(i * blk, blk)] + 1.0)
        dinv = lax.rsqrt(deg)
        out = (a_ref[0] + a_ref[1] + g_ref[...]) * dinv[:, None] + b_ref[...]
        z = jnp.maximum(out, 0.0)
        nrm = jnp.sqrt(jnp.sum(z * z, axis=1, keepdims=True))
        z_ref[...] = z / jnp.maximum(nrm, 1e-12)

    return pl.pallas_call(
        body,
        grid=((n + blk - 1) // blk,),
        in_specs=[
            pl.BlockSpec((2, blk, h_dim), lambda i: (0, i, 0)),
            pl.BlockSpec((blk, h_dim), lambda i: (i, 0)),
            pl.BlockSpec((2, n_pad), lambda i: (0, 0)),
            pl.BlockSpec((1, h_dim), lambda i: (0, 0)),
        ],
        out_specs=pl.BlockSpec((blk, h_dim), lambda i: (i, 0)),
        out_shape=jax.ShapeDtypeStruct((n, h_dim), jnp.float32),
    )(acc2, g, deg2, b2)


def _sc_dots(z, src, dstm1e):
    n, h_dim = z.shape
    e_total = src.shape[0]
    epw = e_total // NW
    C = 128
    CE = C + 8
    nchunks = epw // C
    K = h_dim // LANE

    @functools.partial(
        pl.kernel,
        out_type=(jax.ShapeDtypeStruct((e_total * LANE,), jnp.float32),
                  jax.ShapeDtypeStruct((e_total * LANE,), jnp.float32)),
        mesh=_mesh(),
        scratch_types=[
            pltpu.VMEM((C,), jnp.int32),
            pltpu.VMEM((C,), jnp.int32),
            pltpu.VMEM((CE,), jnp.int32),
            pltpu.VMEM((CE,), jnp.int32),
            pltpu.VMEM((C, h_dim), jnp.float32),
            pltpu.VMEM((C, h_dim), jnp.float32),
            pltpu.VMEM((CE, h_dim), jnp.float32),
            pltpu.VMEM((CE, h_dim), jnp.float32),
            pltpu.VMEM((C * LANE,), jnp.float32),
            pltpu.VMEM((C * LANE,), jnp.float32),
            pltpu.SemaphoreType.DMA,
            pltpu.SemaphoreType.DMA,
        ],
    )
    def dots_kernel(z_hbm, src_hbm, dm1_hbm, pos_hbm, neg_hbm,
                    sidx0_v, sidx1_v, didx0_v, didx1_v, zs0_v, zs1_v,
                    ze0_v, ze1_v, pos_v, neg_v, sem0, sem1):
        c = lax.axis_index("c")
        s = lax.axis_index("s")
        wid = s * NC + c
        ebase = wid * epw
        sems = (sem0, sem1)
        sidxs = (sidx0_v, sidx1_v)
        didxs = (didx0_v, didx1_v)
        zss = (zs0_v, zs1_v)
        zes = (ze0_v, ze1_v)

        def fetch(j, b):
            base = ebase + j * C
            pltpu.sync_copy(src_hbm.at[pl.ds(base, C)], sidxs[b])
            pltpu.sync_copy(dm1_hbm.at[pl.ds(base, CE)], didxs[b])
            pltpu.async_copy(z_hbm.at[sidxs[b]], zss[b], sems[b])
            pltpu.async_copy(z_hbm.at[didxs[b]], zes[b], sems[b])

        def drain(b):
            pltpu.make_async_copy(z_hbm.at[sidxs[b]], zss[b],
                                  sems[b]).wait()
            pltpu.make_async_copy(z_hbm.at[didxs[b]], zes[b],
                                  sems[b]).wait()

        def compute(j, b):
            base = ebase + j * C
            zs_v = zss[b]
            ze_v = zes[b]
            init = tuple(ze_v[0, pl.ds(k * LANE, LANE)] for k in range(K))

            def edge(r, carry):
                acc_p = jnp.zeros((LANE,), jnp.float32)
                acc_q = jnp.zeros((LANE,), jnp.float32)
                nxt = []
                for k in range(K):
                    vs = zs_v[r, pl.ds(k * LANE, LANE)]
                    zn = ze_v[r + 1, pl.ds(k * LANE, LANE)]
                    acc_q = acc_q + vs * carry[k]
                    acc_p = acc_p + vs * zn
                    nxt.append(zn)
                pos_v[pl.ds(r * LANE, LANE)] = acc_p
                neg_v[pl.ds(r * LANE, LANE)] = acc_q
                return tuple(nxt)
            lax.fori_loop(0, C, edge, init)
            pltpu.sync_copy(pos_v, pos_hbm.at[pl.ds(base * LANE, C * LANE)])
            pltpu.sync_copy(neg_v, neg_hbm.at[pl.ds(base * LANE, C * LANE)])

        fetch(0, 0)

        def body(gi, _):
            j0 = gi * 2
            fetch(j0 + 1, 1)
            drain(0)
            compute(j0, 0)
            fetch(lax.rem(j0 + 2, nchunks), 0)
            drain(1)
            compute(j0 + 1, 1)
            return 0
        lax.fori_loop(0, nchunks // 2, body, 0)
        drain(0)
        if nchunks % 2 == 1:
            compute(nchunks - 1, 0)

    return dots_kernel(z, src, dstm1e)


def _tc_loss(pos2, neg2, e_real):
    rows = pos2.shape[0]
    blk = rows // 8
    nsteps = rows // blk
    rows_real = e_real // 8

    def body(p_ref, n_ref, o_ref):
        i = pl.program_id(0)
        ri = lax.broadcasted_iota(jnp.int32, (128, 8), 0) // LANE
        ci = lax.broadcasted_iota(jnp.int32, (128, 8), 1)
        m = (ri == ci).astype(jnp.float32)
        ps = jnp.dot(p_ref[...], m, preferred_element_type=jnp.float32)
        ns = jnp.dot(n_ref[...], m, preferred_element_type=jnp.float32)

        def ls(x):
            return jnp.minimum(x, 0.0) - jnp.log1p(jnp.exp(-jnp.abs(x)))
        rid = i * blk + lax.broadcasted_iota(jnp.int32, (blk, 8), 0)
        valid = (rid < rows_real).astype(jnp.float32)
        part = jnp.sum((ls(ps) + ls(-ns)) * valid)
        prev = jnp.where(i == 0, 0.0, o_ref[0, 0])
        tot = prev + part
        o_ref[0, 0] = jnp.where(i == nsteps - 1, -tot / float(e_real), tot)

    return pl.pallas_call(
        body,
        grid=(nsteps,),
        in_specs=[
            pl.BlockSpec((blk, 128), lambda i: (i, 0)),
            pl.BlockSpec((blk, 128), lambda i: (i, 0)),
        ],
        out_specs=pl.BlockSpec((1, 1), lambda i: (0, 0),
                               memory_space=pltpu.SMEM),
        out_shape=jax.ShapeDtypeStruct((1, 1), jnp.float32),
    )(pos2, neg2)


def kernel(node_features, edge_indices, W, b):
    n, _ = node_features.shape
    h_dim = W.shape[1]
    n_pad = ((n + 2047) // 2048) * 2048
    src = edge_indices[0]
    dst = edge_indices[1]
    e_total = src.shape[0]
    span = 128 * NW
    e_pad = ((e_total + span - 1) // span) * span
    padz = jnp.zeros((e_pad - e_total,), dst.dtype)
    src_p = jnp.concatenate([src, padz])
    dstm1e = jnp.concatenate([dst[-1:], dst, padz, jnp.zeros((7,), dst.dtype)])

    deg2 = _sc_degree(dst, n_pad)
    g = _tc_encode(node_features, W, deg2)
    acc2 = _sc_scatter(g, src, dst, n_pad)
    z = _tc_normalize(acc2, g, deg2, b.reshape(1, h_dim))
    pos, neg = _sc_dots(z, src_p, dstm1e)
    loss = _tc_loss(pos.reshape(-1, 128), neg.reshape(-1, 128), e_total)
    return z, loss[0, 0]


# speedup vs baseline: 1.4760x; 1.4760x over previous
import functools

import jax
import jax.numpy as jnp
from jax import lax
from jax.experimental import pallas as pl
from jax.experimental.pallas import tpu as pltpu
from jax.experimental.pallas import tpu_sc as plsc

NC = 2
NS = 16
NW = NC * NS
LANE = 16


def _mesh():
    return plsc.VectorSubcoreMesh(core_axis_name="c", subcore_axis_name="s")


def _fill_1d(ref, n, value):
    def body(i, _):
        ref[pl.ds(i * LANE, LANE)] = jnp.full((LANE,), value, jnp.float32)
        return 0
    lax.fori_loop(0, n // LANE, body, 0)


def _fill_2d(ref, rows, cols, value):
    def rbody(r, _):
        def cbody(k, _2):
            ref[r, pl.ds(k * LANE, LANE)] = jnp.full((LANE,), value, jnp.float32)
            return 0
        lax.fori_loop(0, cols // LANE, cbody, 0)
        return 0
    lax.fori_loop(0, rows, rbody, 0)


def _sc_degree(dst, n_pad):
    e_total = dst.shape[0]
    epw = e_total // NW
    C = 2000
    nchunks = epw // C
    rpt = n_pad // NS

    @functools.partial(
        pl.kernel,
        out_type=jax.ShapeDtypeStruct((NC, n_pad), jnp.float32),
        mesh=_mesh(),
        scratch_types=[
            pltpu.VMEM((C,), jnp.int32),
            pltpu.VMEM((C,), jnp.float32),
            pltpu.VMEM((rpt,), jnp.float32),
            pltpu.VMEM_SHARED((n_pad,), jnp.float32),
        ],
    )
    def deg_kernel(dst_hbm, out_hbm, idx_v, ones_v, zbuf_v, deg_sh):
        c = lax.axis_index("c")
        s = lax.axis_index("s")
        wid = s * NC + c
        _fill_1d(ones_v, C, 1.0)
        _fill_1d(zbuf_v, rpt, 0.0)
        pltpu.sync_copy(zbuf_v, deg_sh.at[pl.ds(s * rpt, rpt)])
        plsc.subcore_barrier()

        def body(j, _):
            pltpu.sync_copy(dst_hbm.at[pl.ds(wid * epw + j * C, C)], idx_v)
            pltpu.sync_copy(ones_v, deg_sh.at[idx_v], add=True)
            return 0
        lax.fori_loop(0, nchunks, body, 0)
        plsc.subcore_barrier()
        pltpu.sync_copy(deg_sh.at[pl.ds(s * rpt, rpt)],
                        out_hbm.at[c, pl.ds(s * rpt, rpt)])

    return deg_kernel(dst)


def _tc_encode(x, w, deg2):
    n, d = x.shape
    h_dim = w.shape[1]
    blk = 512

    n_pad = deg2.shape[1]

    def body(x_ref, w_ref, deg_ref, g_ref):
        i = pl.program_id(0)
        deg = (deg_ref[0, pl.ds(i * blk, blk)]
               + deg_ref[1, pl.ds(i * blk, blk)] + 1.0)
        dinv = lax.rsqrt(deg)
        h = jnp.dot(x_ref[...], w_ref[...], preferred_element_type=jnp.float32)
        g_ref[...] = h * dinv[:, None]

    return pl.pallas_call(
        body,
        grid=((n + blk - 1) // blk,),
        in_specs=[
            pl.BlockSpec((blk, d), lambda i: (i, 0)),
            pl.BlockSpec((d, h_dim), lambda i: (0, 0)),
            pl.BlockSpec((2, n_pad), lambda i: (0, 0)),
        ],
        out_specs=pl.BlockSpec((blk, h_dim), lambda i: (i, 0)),
        out_shape=jax.ShapeDtypeStruct((n, h_dim), jnp.float32),
    )(x, w, deg2)


def _sc_scatter(g, src, dst, n_pad):
    n, h_dim = g.shape
    e_total = src.shape[0]
    epw = e_total // NW
    C = 80
    nchunks = epw // C
    rpt = n_pad // NS
    zrows = 64

    @functools.partial(
        pl.kernel,
        out_type=jax.ShapeDtypeStruct((NC, n_pad, h_dim), jnp.float32),
        mesh=_mesh(),
        scratch_types=[
            pltpu.VMEM((C,), jnp.int32),
            pltpu.VMEM((C,), jnp.int32),
            pltpu.VMEM((C,), jnp.int32),
            pltpu.VMEM((C,), jnp.int32),
            pltpu.VMEM((C, h_dim), jnp.float32),
            pltpu.VMEM((C, h_dim), jnp.float32),
            pltpu.VMEM((zrows, h_dim), jnp.float32),
            pltpu.VMEM_SHARED((n_pad, h_dim), jnp.float32),
            pltpu.SemaphoreType.DMA,
            pltpu.SemaphoreType.DMA,
        ],
    )
    def scat_kernel(g_hbm, src_hbm, dst_hbm, out_hbm,
                    sidx0_v, sidx1_v, didx0_v, didx1_v, rows0_v, rows1_v,
                    zbuf_v, acc_sh, sem0, sem1):
        c = lax.axis_index("c")
        s = lax.axis_index("s")
        wid = s * NC + c
        ebase = wid * epw
        sems = (sem0, sem1)
        sidxs = (sidx0_v, sidx1_v)
        didxs = (didx0_v, didx1_v)
        rows = (rows0_v, rows1_v)
        _fill_2d(zbuf_v, zrows, h_dim, 0.0)

        def zinit(k, _):
            pltpu.sync_copy(zbuf_v, acc_sh.at[pl.ds(s * rpt + k * zrows, zrows)])
            return 0
        lax.fori_loop(0, rpt // zrows, zinit, 0)
        plsc.subcore_barrier()

        def fetch(j, b):
            base = ebase + j * C
            pltpu.sync_copy(src_hbm.at[pl.ds(base, C)], sidxs[b])
            pltpu.sync_copy(dst_hbm.at[pl.ds(base, C)], didxs[b])
            pltpu.async_copy(g_hbm.at[sidxs[b]], rows[b], sems[b])

        def drain(b):
            pltpu.make_async_copy(g_hbm.at[sidxs[b]], rows[b],
                                  sems[b]).wait()

        def process(b):
            pltpu.sync_copy(rows[b], acc_sh.at[didxs[b]], add=True)

        fetch(0, 0)

        def body(gi, _):
            j0 = gi * 2
            fetch(j0 + 1, 1)
            drain(0)
            process(0)
            fetch(lax.rem(j0 + 2, nchunks), 0)
            drain(1)
            process(1)
            return 0
        lax.fori_loop(0, nchunks // 2, body, 0)
        drain(0)
        if nchunks % 2 == 1:
            process(0)
        plsc.subcore_barrier()

        def wback(k, _):
            off = s * rpt + k * zrows
            pltpu.sync_copy(acc_sh.at[pl.ds(off, zrows)],
                            out_hbm.at[c, pl.ds(off, zrows)])
            return 0
        lax.fori_loop(0, rpt // zrows, wback, 0)

    return scat_kernel(g, src, dst)


def _tc_normalize(acc2, g, deg2, b2):
    n, h_dim = g.shape
    blk = 512

    n_pad = deg2.shape[1]

    def body(a_ref, g_ref, deg_ref, b_ref, z_ref):
        i = pl.program_id(0)
        deg = (deg_ref[0, pl.ds(i * blk, blk)]
               + deg_ref[1, pl.ds(i * blk, blk)] + 1.0)
        dinv = lax.rsqrt(deg)
        out = (a_ref[0] + a_ref[1] + g_ref[...]) * dinv[:, None] + b_ref[...]
        z = jnp.maximum(out, 0.0)
        nrm = jnp.sqrt(jnp.sum(z * z, axis=1, keepdims=True))
        z_ref[...] = z / jnp.maximum(nrm, 1e-12)

    return pl.pallas_call(
        body,
        grid=((n + blk - 1) // blk,),
        in_specs=[
            pl.BlockSpec((2, blk, h_dim), lambda i: (0, i, 0)),
            pl.BlockSpec((blk, h_dim), lambda i: (i, 0)),
            pl.BlockSpec((2, n_pad), lambda i: (0, 0)),
            pl.BlockSpec((1, h_dim), lambda i: (0, 0)),
        ],
        out_specs=pl.BlockSpec((blk, h_dim), lambda i: (i, 0)),
        out_shape=jax.ShapeDtypeStruct((n, h_dim), jnp.float32),
    )(acc2, g, deg2, b2)


def _sc_dots(z, src, dstm1e):
    n, h_dim = z.shape
    e_total = src.shape[0]
    epw = e_total // NW
    C = 80
    CE = C + 8
    nchunks = epw // C
    K = h_dim // LANE

    @functools.partial(
        pl.kernel,
        out_type=(jax.ShapeDtypeStruct((e_total * LANE,), jnp.float32),
                  jax.ShapeDtypeStruct((e_total * LANE,), jnp.float32)),
        mesh=_mesh(),
        scratch_types=[
            pltpu.VMEM((C,), jnp.int32),
            pltpu.VMEM((C,), jnp.int32),
            pltpu.VMEM((CE,), jnp.int32),
            pltpu.VMEM((CE,), jnp.int32),
            pltpu.VMEM((C, h_dim), jnp.float32),
            pltpu.VMEM((C, h_dim), jnp.float32),
            pltpu.VMEM((CE, h_dim), jnp.float32),
            pltpu.VMEM((CE, h_dim), jnp.float32),
            pltpu.VMEM((C * LANE,), jnp.float32),
            pltpu.VMEM((C * LANE,), jnp.float32),
            pltpu.SemaphoreType.DMA,
            pltpu.SemaphoreType.DMA,
        ],
    )
    def dots_kernel(z_hbm, src_hbm, dm1_hbm, pos_hbm, neg_hbm,
                    sidx0_v, sidx1_v, didx0_v, didx1_v, zs0_v, zs1_v,
                    ze0_v, ze1_v, pos_v, neg_v, sem0, sem1):
        c = lax.axis_index("c")
        s = lax.axis_index("s")
        wid = s * NC + c
        ebase = wid * epw
        sems = (sem0, sem1)
        sidxs = (sidx0_v, sidx1_v)
        didxs = (didx0_v, didx1_v)
        zss = (zs0_v, zs1_v)
        zes = (ze0_v, ze1_v)

        def fetch(j, b):
            base = ebase + j * C
            pltpu.sync_copy(src_hbm.at[pl.ds(base, C)], sidxs[b])
            pltpu.sync_copy(dm1_hbm.at[pl.ds(base, CE)], didxs[b])
            pltpu.async_copy(z_hbm.at[sidxs[b]], zss[b], sems[b])
            pltpu.async_copy(z_hbm.at[didxs[b]], zes[b], sems[b])

        def drain(b):
            pltpu.make_async_copy(z_hbm.at[sidxs[b]], zss[b],
                                  sems[b]).wait()
            pltpu.make_async_copy(z_hbm.at[didxs[b]], zes[b],
                                  sems[b]).wait()

        def compute(j, b):
            base = ebase + j * C
            zs_v = zss[b]
            ze_v = zes[b]
            init = tuple(ze_v[0, pl.ds(k * LANE, LANE)] for k in range(K))

            def edge(r, carry):
                acc_p = jnp.zeros((LANE,), jnp.float32)
                acc_q = jnp.zeros((LANE,), jnp.float32)
                nxt = []
                for k in range(K):
                    vs = zs_v[r, pl.ds(k * LANE, LANE)]
                    zn = ze_v[r + 1, pl.ds(k * LANE, LANE)]
                    acc_q = acc_q + vs * carry[k]
                    acc_p = acc_p + vs * zn
                    nxt.append(zn)
                pos_v[pl.ds(r * LANE, LANE)] = acc_p
                neg_v[pl.ds(r * LANE, LANE)] = acc_q
                return tuple(nxt)
            lax.fori_loop(0, C, edge, init)
            pltpu.sync_copy(pos_v, pos_hbm.at[pl.ds(base * LANE, C * LANE)])
            pltpu.sync_copy(neg_v, neg_hbm.at[pl.ds(base * LANE, C * LANE)])

        fetch(0, 0)

        def body(gi, _):
            j0 = gi * 2
            fetch(j0 + 1, 1)
            drain(0)
            compute(j0, 0)
            fetch(lax.rem(j0 + 2, nchunks), 0)
            drain(1)
            compute(j0 + 1, 1)
            return 0
        lax.fori_loop(0, nchunks // 2, body, 0)
        drain(0)
        if nchunks % 2 == 1:
            compute(nchunks - 1, 0)

    return dots_kernel(z, src, dstm1e)


def _tc_loss(pos2, neg2, e_real):
    rows = pos2.shape[0]
    blk = rows // 8
    nsteps = rows // blk
    rows_real = e_real // 8

    def body(p_ref, n_ref, o_ref):
        i = pl.program_id(0)
        ri = lax.broadcasted_iota(jnp.int32, (128, 8), 0) // LANE
        ci = lax.broadcasted_iota(jnp.int32, (128, 8), 1)
        m = (ri == ci).astype(jnp.float32)
        ps = jnp.dot(p_ref[...], m, preferred_element_type=jnp.float32)
        ns = jnp.dot(n_ref[...], m, preferred_element_type=jnp.float32)

        def ls(x):
            return jnp.minimum(x, 0.0) - jnp.log1p(jnp.exp(-jnp.abs(x)))
        rid = i * blk + lax.broadcasted_iota(jnp.int32, (blk, 8), 0)
        valid = (rid < rows_real).astype(jnp.float32)
        part = jnp.sum((ls(ps) + ls(-ns)) * valid)
        prev = jnp.where(i == 0, 0.0, o_ref[0, 0])
        tot = prev + part
        o_ref[0, 0] = jnp.where(i == nsteps - 1, -tot / float(e_real), tot)

    return pl.pallas_call(
        body,
        grid=(nsteps,),
        in_specs=[
            pl.BlockSpec((blk, 128), lambda i: (i, 0)),
            pl.BlockSpec((blk, 128), lambda i: (i, 0)),
        ],
        out_specs=pl.BlockSpec((1, 1), lambda i: (0, 0),
                               memory_space=pltpu.SMEM),
        out_shape=jax.ShapeDtypeStruct((1, 1), jnp.float32),
    )(pos2, neg2)


def kernel(node_features, edge_indices, W, b):
    n, _ = node_features.shape
    h_dim = W.shape[1]
    n_pad = ((n + 2047) // 2048) * 2048
    src = edge_indices[0]
    dst = edge_indices[1]
    e_total = src.shape[0]
    dstm1e = jnp.concatenate([dst[-1:], dst, jnp.zeros((7,), dst.dtype)])

    deg2 = _sc_degree(dst, n_pad)
    g = _tc_encode(node_features, W, deg2)
    acc2 = _sc_scatter(g, src, dst, n_pad)
    z = _tc_normalize(acc2, g, deg2, b.reshape(1, h_dim))
    pos, neg = _sc_dots(z, src, dstm1e)
    loss = _tc_loss(pos.reshape(-1, 128), neg.reshape(-1, 128), e_total)
    return z, loss[0, 0]


# speedup vs baseline: 1.4863x; 1.0070x over previous
import functools

import jax
import jax.numpy as jnp
from jax import lax
from jax.experimental import pallas as pl
from jax.experimental.pallas import tpu as pltpu
from jax.experimental.pallas import tpu_sc as plsc

NC = 2
NS = 16
NW = NC * NS
LANE = 16


def _mesh():
    return plsc.VectorSubcoreMesh(core_axis_name="c", subcore_axis_name="s")


def _fill_1d(ref, n, value):
    def body(i, _):
        ref[pl.ds(i * LANE, LANE)] = jnp.full((LANE,), value, jnp.float32)
        return 0
    lax.fori_loop(0, n // LANE, body, 0)


def _fill_2d(ref, rows, cols, value):
    def rbody(r, _):
        def cbody(k, _2):
            ref[r, pl.ds(k * LANE, LANE)] = jnp.full((LANE,), value, jnp.float32)
            return 0
        lax.fori_loop(0, cols // LANE, cbody, 0)
        return 0
    lax.fori_loop(0, rows, rbody, 0)


def _sc_degree(dst, n_pad):
    e_total = dst.shape[0]
    epw = e_total // NW
    C = 2000
    nchunks = epw // C
    rpt = n_pad // NS

    @functools.partial(
        pl.kernel,
        out_type=jax.ShapeDtypeStruct((NC, n_pad), jnp.float32),
        mesh=_mesh(),
        scratch_types=[
            pltpu.VMEM((C,), jnp.int32),
            pltpu.VMEM((C,), jnp.float32),
            pltpu.VMEM((rpt,), jnp.float32),
            pltpu.VMEM_SHARED((n_pad,), jnp.float32),
        ],
    )
    def deg_kernel(dst_hbm, out_hbm, idx_v, ones_v, zbuf_v, deg_sh):
        c = lax.axis_index("c")
        s = lax.axis_index("s")
        wid = s * NC + c
        _fill_1d(ones_v, C, 1.0)
        _fill_1d(zbuf_v, rpt, 0.0)
        pltpu.sync_copy(zbuf_v, deg_sh.at[pl.ds(s * rpt, rpt)])
        plsc.subcore_barrier()

        def body(j, _):
            pltpu.sync_copy(dst_hbm.at[pl.ds(wid * epw + j * C, C)], idx_v)
            pltpu.sync_copy(ones_v, deg_sh.at[idx_v], add=True)
            return 0
        lax.fori_loop(0, nchunks, body, 0)
        plsc.subcore_barrier()
        pltpu.sync_copy(deg_sh.at[pl.ds(s * rpt, rpt)],
                        out_hbm.at[c, pl.ds(s * rpt, rpt)])

    return deg_kernel(dst)


def _tc_encode(x, w, deg2):
    n, d = x.shape
    h_dim = w.shape[1]
    blk = 512

    n_pad = deg2.shape[1]

    def body(x_ref, w_ref, deg_ref, g_ref):
        i = pl.program_id(0)
        deg = (deg_ref[0, pl.ds(i * blk, blk)]
               + deg_ref[1, pl.ds(i * blk, blk)] + 1.0)
        dinv = lax.rsqrt(deg)
        h = jnp.dot(x_ref[...], w_ref[...], preferred_element_type=jnp.float32)
        g_ref[...] = h * dinv[:, None]

    return pl.pallas_call(
        body,
        grid=((n + blk - 1) // blk,),
        in_specs=[
            pl.BlockSpec((blk, d), lambda i: (i, 0)),
            pl.BlockSpec((d, h_dim), lambda i: (0, 0)),
            pl.BlockSpec((2, n_pad), lambda i: (0, 0)),
        ],
        out_specs=pl.BlockSpec((blk, h_dim), lambda i: (i, 0)),
        out_shape=jax.ShapeDtypeStruct((n, h_dim), jnp.float32),
    )(x, w, deg2)


def _sc_scatter(g, src, dst, n_pad):
    n, h_dim = g.shape
    e_total = src.shape[0]
    epw = e_total // NW
    C = 80
    nchunks = epw // C
    rpt = n_pad // NS
    zrows = 64

    @functools.partial(
        pl.kernel,
        out_type=jax.ShapeDtypeStruct((NC, n_pad, h_dim), jnp.float32),
        mesh=_mesh(),
        scratch_types=[
            pltpu.VMEM((C,), jnp.int32),
            pltpu.VMEM((C,), jnp.int32),
            pltpu.VMEM((C,), jnp.int32),
            pltpu.VMEM((C,), jnp.int32),
            pltpu.VMEM((C, h_dim), jnp.float32),
            pltpu.VMEM((C, h_dim), jnp.float32),
            pltpu.VMEM((zrows, h_dim), jnp.float32),
            pltpu.VMEM_SHARED((n_pad, h_dim), jnp.float32),
            pltpu.SemaphoreType.DMA,
            pltpu.SemaphoreType.DMA,
        ],
    )
    def scat_kernel(g_hbm, src_hbm, dst_hbm, out_hbm,
                    sidx0_v, sidx1_v, didx0_v, didx1_v, rows0_v, rows1_v,
                    zbuf_v, acc_sh, sem0, sem1):
        c = lax.axis_index("c")
        s = lax.axis_index("s")
        wid = s * NC + c
        ebase = wid * epw
        sems = (sem0, sem1)
        sidxs = (sidx0_v, sidx1_v)
        didxs = (didx0_v, didx1_v)
        rows = (rows0_v, rows1_v)
        _fill_2d(zbuf_v, zrows, h_dim, 0.0)

        def zinit(k, _):
            pltpu.sync_copy(zbuf_v, acc_sh.at[pl.ds(s * rpt + k * zrows, zrows)])
            return 0
        lax.fori_loop(0, rpt // zrows, zinit, 0)
        plsc.subcore_barrier()

        def fetch(j, b):
            base = ebase + j * C
            pltpu.sync_copy(src_hbm.at[pl.ds(base, C)], sidxs[b])
            pltpu.sync_copy(dst_hbm.at[pl.ds(base, C)], didxs[b])
            pltpu.async_copy(g_hbm.at[sidxs[b]], rows[b], sems[b])

        def drain(b):
            pltpu.make_async_copy(g_hbm.at[sidxs[b]], rows[b],
                                  sems[b]).wait()

        def process(b):
            pltpu.sync_copy(rows[b], acc_sh.at[didxs[b]], add=True)

        fetch(0, 0)

        def body(gi, _):
            j0 = gi * 2
            fetch(j0 + 1, 1)
            drain(0)
            process(0)
            fetch(lax.rem(j0 + 2, nchunks), 0)
            drain(1)
            process(1)
            return 0
        lax.fori_loop(0, nchunks // 2, body, 0)
        drain(0)
        if nchunks % 2 == 1:
            process(0)
        plsc.subcore_barrier()

        def wback(k, _):
            off = s * rpt + k * zrows
            pltpu.sync_copy(acc_sh.at[pl.ds(off, zrows)],
                            out_hbm.at[c, pl.ds(off, zrows)])
            return 0
        lax.fori_loop(0, rpt // zrows, wback, 0)

    return scat_kernel(g, src, dst)


def _tc_normalize(acc2, g, deg2, b2):
    n, h_dim = g.shape
    blk = 512

    n_pad = deg2.shape[1]

    def body(a_ref, g_ref, deg_ref, b_ref, z_ref):
        i = pl.program_id(0)
        deg = (deg_ref[0, pl.ds(i * blk, blk)]
               + deg_ref[1, pl.ds(i * blk, blk)] + 1.0)
        dinv = lax.rsqrt(deg)
        out = (a_ref[0] + a_ref[1] + g_ref[...]) * dinv[:, None] + b_ref[...]
        z = jnp.maximum(out, 0.0)
        nrm = jnp.sqrt(jnp.sum(z * z, axis=1, keepdims=True))
        z_ref[...] = z / jnp.maximum(nrm, 1e-12)

    return pl.pallas_call(
        body,
        grid=((n + blk - 1) // blk,),
        in_specs=[
            pl.BlockSpec((2, blk, h_dim), lambda i: (0, i, 0)),
            pl.BlockSpec((blk, h_dim), lambda i: (i, 0)),
            pl.BlockSpec((2, n_pad), lambda i: (0, 0)),
            pl.BlockSpec((1, h_dim), lambda i: (0, 0)),
        ],
        out_specs=pl.BlockSpec((blk, h_dim), lambda i: (i, 0)),
        out_shape=jax.ShapeDtypeStruct((n, h_dim), jnp.float32),
    )(acc2, g, deg2, b2)


def _sc_dots(z, src, dstm1e):
    n, h_dim = z.shape
    e_total = src.shape[0]
    epw = e_total // NW
    C = 80
    CE = C + 8
    nchunks = epw // C
    K = h_dim // LANE

    @functools.partial(
        pl.kernel,
        out_type=(jax.ShapeDtypeStruct((e_total * LANE,), jnp.float32),
                  jax.ShapeDtypeStruct((e_total * LANE,), jnp.float32)),
        mesh=_mesh(),
        scratch_types=[
            pltpu.VMEM((C,), jnp.int32),
            pltpu.VMEM((C,), jnp.int32),
            pltpu.VMEM((CE,), jnp.int32),
            pltpu.VMEM((CE,), jnp.int32),
            pltpu.VMEM((C, h_dim), jnp.float32),
            pltpu.VMEM((C, h_dim), jnp.float32),
            pltpu.VMEM((CE, h_dim), jnp.float32),
            pltpu.VMEM((CE, h_dim), jnp.float32),
            pltpu.VMEM((C * LANE,), jnp.float32),
            pltpu.VMEM((C * LANE,), jnp.float32),
            pltpu.SemaphoreType.DMA,
            pltpu.SemaphoreType.DMA,
        ],
    )
    def dots_kernel(z_hbm, src_hbm, dm1_hbm, pos_hbm, neg_hbm,
                    sidx0_v, sidx1_v, didx0_v, didx1_v, zs0_v, zs1_v,
                    ze0_v, ze1_v, pos_v, neg_v, sem0, sem1):
        c = lax.axis_index("c")
        s = lax.axis_index("s")
        wid = s * NC + c
        ebase = wid * epw
        sems = (sem0, sem1)
        sidxs = (sidx0_v, sidx1_v)
        didxs = (didx0_v, didx1_v)
        zss = (zs0_v, zs1_v)
        zes = (ze0_v, ze1_v)

        def fetch(j, b):
            base = ebase + j * C
            pltpu.sync_copy(src_hbm.at[pl.ds(base, C)], sidxs[b])
            pltpu.sync_copy(dm1_hbm.at[pl.ds(base, CE)], didxs[b])
            pltpu.async_copy(z_hbm.at[sidxs[b]], zss[b], sems[b])
            pltpu.async_copy(z_hbm.at[didxs[b]], zes[b], sems[b])

        def drain(b):
            pltpu.make_async_copy(z_hbm.at[sidxs[b]], zss[b],
                                  sems[b]).wait()
            pltpu.make_async_copy(z_hbm.at[didxs[b]], zes[b],
                                  sems[b]).wait()

        def compute(j, b):
            base = ebase + j * C
            zs_v = zss[b]
            ze_v = zes[b]
            init = tuple(ze_v[0, pl.ds(k * LANE, LANE)] for k in range(K))

            def edge(r, carry):
                acc_p = jnp.zeros((LANE,), jnp.float32)
                acc_q = jnp.zeros((LANE,), jnp.float32)
                nxt = []
                for k in range(K):
                    vs = zs_v[r, pl.ds(k * LANE, LANE)]
                    zn = ze_v[r + 1, pl.ds(k * LANE, LANE)]
                    acc_q = acc_q + vs * carry[k]
                    acc_p = acc_p + vs * zn
                    nxt.append(zn)
                pos_v[pl.ds(r * LANE, LANE)] = acc_p
                neg_v[pl.ds(r * LANE, LANE)] = acc_q
                return tuple(nxt)
            lax.fori_loop(0, C, edge, init, unroll=4)
            pltpu.sync_copy(pos_v, pos_hbm.at[pl.ds(base * LANE, C * LANE)])
            pltpu.sync_copy(neg_v, neg_hbm.at[pl.ds(base * LANE, C * LANE)])

        fetch(0, 0)

        def body(gi, _):
            j0 = gi * 2
            fetch(j0 + 1, 1)
            drain(0)
            compute(j0, 0)
            fetch(lax.rem(j0 + 2, nchunks), 0)
            drain(1)
            compute(j0 + 1, 1)
            return 0
        lax.fori_loop(0, nchunks // 2, body, 0)
        drain(0)
        if nchunks % 2 == 1:
            compute(nchunks - 1, 0)

    return dots_kernel(z, src, dstm1e)


def _tc_loss(pos2, neg2, e_real):
    rows = pos2.shape[0]
    blk = rows // 8
    nsteps = rows // blk
    rows_real = e_real // 8

    def body(p_ref, n_ref, o_ref):
        i = pl.program_id(0)
        ri = lax.broadcasted_iota(jnp.int32, (128, 8), 0) // LANE
        ci = lax.broadcasted_iota(jnp.int32, (128, 8), 1)
        m = (ri == ci).astype(jnp.float32)
        ps = jnp.dot(p_ref[...], m, preferred_element_type=jnp.float32)
        ns = jnp.dot(n_ref[...], m, preferred_element_type=jnp.float32)

        def ls(x):
            return jnp.minimum(x, 0.0) - jnp.log1p(jnp.exp(-jnp.abs(x)))
        rid = i * blk + lax.broadcasted_iota(jnp.int32, (blk, 8), 0)
        valid = (rid < rows_real).astype(jnp.float32)
        part = jnp.sum((ls(ps) + ls(-ns)) * valid)
        prev = jnp.where(i == 0, 0.0, o_ref[0, 0])
        tot = prev + part
        o_ref[0, 0] = jnp.where(i == nsteps - 1, -tot / float(e_real), tot)

    return pl.pallas_call(
        body,
        grid=(nsteps,),
        in_specs=[
            pl.BlockSpec((blk, 128), lambda i: (i, 0)),
            pl.BlockSpec((blk, 128), lambda i: (i, 0)),
        ],
        out_specs=pl.BlockSpec((1, 1), lambda i: (0, 0),
                               memory_space=pltpu.SMEM),
        out_shape=jax.ShapeDtypeStruct((1, 1), jnp.float32),
    )(pos2, neg2)


def kernel(node_features, edge_indices, W, b):
    n, _ = node_features.shape
    h_dim = W.shape[1]
    n_pad = ((n + 2047) // 2048) * 2048
    src = edge_indices[0]
    dst = edge_indices[1]
    e_total = src.shape[0]
    dstm1e = jnp.concatenate([dst[-1:], dst, jnp.zeros((7,), dst.dtype)])

    deg2 = _sc_degree(dst, n_pad)
    g = _tc_encode(node_features, W, deg2)
    acc2 = _sc_scatter(g, src, dst, n_pad)
    z = _tc_normalize(acc2, g, deg2, b.reshape(1, h_dim))
    pos, neg = _sc_dots(z, src, dstm1e)
    loss = _tc_loss(pos.reshape(-1, 128), neg.reshape(-1, 128), e_total)
    return z, loss[0, 0]


# speedup vs baseline: 1.6800x; 1.1303x over previous
import functools

import jax
import jax.numpy as jnp
from jax import lax
from jax.experimental import pallas as pl
from jax.experimental.pallas import tpu as pltpu
from jax.experimental.pallas import tpu_sc as plsc

NC = 2
NS = 16
NW = NC * NS
LANE = 16


def _mesh():
    return plsc.VectorSubcoreMesh(core_axis_name="c", subcore_axis_name="s")


def _fill_1d(ref, n, value):
    def body(i, _):
        ref[pl.ds(i * LANE, LANE)] = jnp.full((LANE,), value, jnp.float32)
        return 0
    lax.fori_loop(0, n // LANE, body, 0)


def _fill_2d(ref, rows, cols, value):
    def rbody(r, _):
        def cbody(k, _2):
            ref[r, pl.ds(k * LANE, LANE)] = jnp.full((LANE,), value, jnp.float32)
            return 0
        lax.fori_loop(0, cols // LANE, cbody, 0)
        return 0
    lax.fori_loop(0, rows, rbody, 0)


def _sc_degree(dst, n_pad):
    e_total = dst.shape[0]
    epw = e_total // NW
    C = 2000
    nchunks = epw // C
    rpt = n_pad // NS

    @functools.partial(
        pl.kernel,
        out_type=jax.ShapeDtypeStruct((NC, n_pad), jnp.float32),
        mesh=_mesh(),
        scratch_types=[
            pltpu.VMEM((C,), jnp.int32),
            pltpu.VMEM((C,), jnp.float32),
            pltpu.VMEM((rpt,), jnp.float32),
            pltpu.VMEM_SHARED((n_pad,), jnp.float32),
        ],
    )
    def deg_kernel(dst_hbm, out_hbm, idx_v, ones_v, zbuf_v, deg_sh):
        c = lax.axis_index("c")
        s = lax.axis_index("s")
        wid = s * NC + c
        _fill_1d(ones_v, C, 1.0)
        _fill_1d(zbuf_v, rpt, 0.0)
        pltpu.sync_copy(zbuf_v, deg_sh.at[pl.ds(s * rpt, rpt)])
        plsc.subcore_barrier()

        def body(j, _):
            pltpu.sync_copy(dst_hbm.at[pl.ds(wid * epw + j * C, C)], idx_v)
            pltpu.sync_copy(ones_v, deg_sh.at[idx_v], add=True)
            return 0
        lax.fori_loop(0, nchunks, body, 0)
        plsc.subcore_barrier()
        pltpu.sync_copy(deg_sh.at[pl.ds(s * rpt, rpt)],
                        out_hbm.at[c, pl.ds(s * rpt, rpt)])

    return deg_kernel(dst)


def _tc_encode(x, w, deg2):
    n, d = x.shape
    h_dim = w.shape[1]
    blk = 512

    n_pad = deg2.shape[1]

    def body(x_ref, w_ref, deg_ref, g_ref):
        i = pl.program_id(0)
        deg = (deg_ref[0, pl.ds(i * blk, blk)]
               + deg_ref[1, pl.ds(i * blk, blk)] + 1.0)
        dinv = lax.rsqrt(deg)
        h = jnp.dot(x_ref[...], w_ref[...], preferred_element_type=jnp.float32)
        g_ref[...] = h * dinv[:, None]

    return pl.pallas_call(
        body,
        grid=((n + blk - 1) // blk,),
        in_specs=[
            pl.BlockSpec((blk, d), lambda i: (i, 0)),
            pl.BlockSpec((d, h_dim), lambda i: (0, 0)),
            pl.BlockSpec((2, n_pad), lambda i: (0, 0)),
        ],
        out_specs=pl.BlockSpec((blk, h_dim), lambda i: (i, 0)),
        out_shape=jax.ShapeDtypeStruct((n, h_dim), jnp.float32),
    )(x, w, deg2)


def _sc_scatter(g, src, dst, n_pad):
    n, h_dim = g.shape
    e_total = src.shape[0]
    epw = e_total // NW
    C = 80
    nchunks = epw // C
    rpt = n_pad // NS
    zrows = 64

    @functools.partial(
        pl.kernel,
        out_type=jax.ShapeDtypeStruct((NC, n_pad, h_dim), jnp.float32),
        mesh=_mesh(),
        scratch_types=[
            pltpu.VMEM((C,), jnp.int32),
            pltpu.VMEM((C,), jnp.int32),
            pltpu.VMEM((C,), jnp.int32),
            pltpu.VMEM((C,), jnp.int32),
            pltpu.VMEM((C, h_dim), jnp.float32),
            pltpu.VMEM((C, h_dim), jnp.float32),
            pltpu.VMEM((zrows, h_dim), jnp.float32),
            pltpu.VMEM_SHARED((n_pad, h_dim), jnp.float32),
            pltpu.SemaphoreType.DMA,
            pltpu.SemaphoreType.DMA,
        ],
    )
    def scat_kernel(g_hbm, src_hbm, dst_hbm, out_hbm,
                    sidx0_v, sidx1_v, didx0_v, didx1_v, rows0_v, rows1_v,
                    zbuf_v, acc_sh, sem0, sem1):
        c = lax.axis_index("c")
        s = lax.axis_index("s")
        wid = s * NC + c
        ebase = wid * epw
        sems = (sem0, sem1)
        sidxs = (sidx0_v, sidx1_v)
        didxs = (didx0_v, didx1_v)
        rows = (rows0_v, rows1_v)
        _fill_2d(zbuf_v, zrows, h_dim, 0.0)

        def zinit(k, _):
            pltpu.sync_copy(zbuf_v, acc_sh.at[pl.ds(s * rpt + k * zrows, zrows)])
            return 0
        lax.fori_loop(0, rpt // zrows, zinit, 0)
        plsc.subcore_barrier()

        def fetch(j, b):
            base = ebase + j * C
            pltpu.sync_copy(src_hbm.at[pl.ds(base, C)], sidxs[b])
            pltpu.sync_copy(dst_hbm.at[pl.ds(base, C)], didxs[b])
            pltpu.async_copy(g_hbm.at[sidxs[b]], rows[b], sems[b])

        def drain(b):
            pltpu.make_async_copy(g_hbm.at[sidxs[b]], rows[b],
                                  sems[b]).wait()

        def process(b):
            pltpu.sync_copy(rows[b], acc_sh.at[didxs[b]], add=True)

        fetch(0, 0)

        def body(gi, _):
            j0 = gi * 2
            fetch(j0 + 1, 1)
            drain(0)
            process(0)
            fetch(lax.rem(j0 + 2, nchunks), 0)
            drain(1)
            process(1)
            return 0
        lax.fori_loop(0, nchunks // 2, body, 0)
        drain(0)
        if nchunks % 2 == 1:
            process(0)
        plsc.subcore_barrier()

        def wback(k, _):
            off = s * rpt + k * zrows
            pltpu.sync_copy(acc_sh.at[pl.ds(off, zrows)],
                            out_hbm.at[c, pl.ds(off, zrows)])
            return 0
        lax.fori_loop(0, rpt // zrows, wback, 0)

    return scat_kernel(g, src, dst)


def _tc_normalize(acc2, g, deg2, b2):
    n, h_dim = g.shape
    blk = 512

    n_pad = deg2.shape[1]

    def body(a_ref, g_ref, deg_ref, b_ref, z_ref):
        i = pl.program_id(0)
        deg = (deg_ref[0, pl.ds(i * blk, blk)]
               + deg_ref[1, pl.ds(i * blk, blk)] + 1.0)
        dinv = lax.rsqrt(deg)
        out = (a_ref[0] + a_ref[1] + g_ref[...]) * dinv[:, None] + b_ref[...]
        z = jnp.maximum(out, 0.0)
        nrm = jnp.sqrt(jnp.sum(z * z, axis=1, keepdims=True))
        z_ref[...] = z / jnp.maximum(nrm, 1e-12)

    return pl.pallas_call(
        body,
        grid=((n + blk - 1) // blk,),
        in_specs=[
            pl.BlockSpec((2, blk, h_dim), lambda i: (0, i, 0)),
            pl.BlockSpec((blk, h_dim), lambda i: (i, 0)),
            pl.BlockSpec((2, n_pad), lambda i: (0, 0)),
            pl.BlockSpec((1, h_dim), lambda i: (0, 0)),
        ],
        out_specs=pl.BlockSpec((blk, h_dim), lambda i: (i, 0)),
        out_shape=jax.ShapeDtypeStruct((n, h_dim), jnp.float32),
    )(acc2, g, deg2, b2)


def _sc_dots(z, src, dstm1e):
    n, h_dim = z.shape
    e_total = src.shape[0]
    epw = e_total // NW
    C = 200
    CE = C + 8
    nchunks = epw // C
    K = h_dim // LANE

    @functools.partial(
        pl.kernel,
        out_type=(jax.ShapeDtypeStruct((e_total * LANE,), jnp.float32),
                  jax.ShapeDtypeStruct((e_total * LANE,), jnp.float32)),
        mesh=_mesh(),
        scratch_types=[
            pltpu.VMEM((C,), jnp.int32),
            pltpu.VMEM((C,), jnp.int32),
            pltpu.VMEM((CE,), jnp.int32),
            pltpu.VMEM((CE,), jnp.int32),
            pltpu.VMEM((C, h_dim), jnp.float32),
            pltpu.VMEM((C, h_dim), jnp.float32),
            pltpu.VMEM((CE, h_dim), jnp.float32),
            pltpu.VMEM((CE, h_dim), jnp.float32),
            pltpu.VMEM((C * LANE,), jnp.float32),
            pltpu.VMEM((C * LANE,), jnp.float32),
            pltpu.SemaphoreType.DMA,
            pltpu.SemaphoreType.DMA,
        ],
    )
    def dots_kernel(z_hbm, src_hbm, dm1_hbm, pos_hbm, neg_hbm,
                    sidx0_v, sidx1_v, didx0_v, didx1_v, zs0_v, zs1_v,
                    ze0_v, ze1_v, pos_v, neg_v, sem0, sem1):
        c = lax.axis_index("c")
        s = lax.axis_index("s")
        wid = s * NC + c
        ebase = wid * epw
        sems = (sem0, sem1)
        sidxs = (sidx0_v, sidx1_v)
        didxs = (didx0_v, didx1_v)
        zss = (zs0_v, zs1_v)
        zes = (ze0_v, ze1_v)

        def fetch(j, b):
            base = ebase + j * C
            pltpu.sync_copy(src_hbm.at[pl.ds(base, C)], sidxs[b])
            pltpu.sync_copy(dm1_hbm.at[pl.ds(base, CE)], didxs[b])
            pltpu.async_copy(z_hbm.at[sidxs[b]], zss[b], sems[b])
            pltpu.async_copy(z_hbm.at[didxs[b]], zes[b], sems[b])

        def drain(b):
            pltpu.make_async_copy(z_hbm.at[sidxs[b]], zss[b],
                                  sems[b]).wait()
            pltpu.make_async_copy(z_hbm.at[didxs[b]], zes[b],
                                  sems[b]).wait()

        def compute(j, b):
            base = ebase + j * C
            zs_v = zss[b]
            ze_v = zes[b]
            init = tuple(ze_v[0, pl.ds(k * LANE, LANE)] for k in range(K))

            def edge(r, carry):
                acc_p = jnp.zeros((LANE,), jnp.float32)
                acc_q = jnp.zeros((LANE,), jnp.float32)
                nxt = []
                for k in range(K):
                    vs = zs_v[r, pl.ds(k * LANE, LANE)]
                    zn = ze_v[r + 1, pl.ds(k * LANE, LANE)]
                    acc_q = acc_q + vs * carry[k]
                    acc_p = acc_p + vs * zn
                    nxt.append(zn)
                pos_v[pl.ds(r * LANE, LANE)] = acc_p
                neg_v[pl.ds(r * LANE, LANE)] = acc_q
                return tuple(nxt)
            lax.fori_loop(0, C, edge, init, unroll=4)
            pltpu.sync_copy(pos_v, pos_hbm.at[pl.ds(base * LANE, C * LANE)])
            pltpu.sync_copy(neg_v, neg_hbm.at[pl.ds(base * LANE, C * LANE)])

        fetch(0, 0)

        def body(gi, _):
            j0 = gi * 2
            fetch(j0 + 1, 1)
            drain(0)
            compute(j0, 0)
            fetch(lax.rem(j0 + 2, nchunks), 0)
            drain(1)
            compute(j0 + 1, 1)
            return 0
        lax.fori_loop(0, nchunks // 2, body, 0)
        drain(0)
        if nchunks % 2 == 1:
            compute(nchunks - 1, 0)

    return dots_kernel(z, src, dstm1e)


def _tc_loss(pos2, neg2, e_real):
    rows = pos2.shape[0]
    blk = rows // 8
    nsteps = rows // blk
    rows_real = e_real // 8

    def body(p_ref, n_ref, o_ref):
        i = pl.program_id(0)
        ri = lax.broadcasted_iota(jnp.int32, (128, 8), 0) // LANE
        ci = lax.broadcasted_iota(jnp.int32, (128, 8), 1)
        m = (ri == ci).astype(jnp.float32)
        ps = jnp.dot(p_ref[...], m, preferred_element_type=jnp.float32)
        ns = jnp.dot(n_ref[...], m, preferred_element_type=jnp.float32)

        def ls(x):
            return jnp.minimum(x, 0.0) - jnp.log1p(jnp.exp(-jnp.abs(x)))
        rid = i * blk + lax.broadcasted_iota(jnp.int32, (blk, 8), 0)
        valid = (rid < rows_real).astype(jnp.float32)
        part = jnp.sum((ls(ps) + ls(-ns)) * valid)
        prev = jnp.where(i == 0, 0.0, o_ref[0, 0])
        tot = prev + part
        o_ref[0, 0] = jnp.where(i == nsteps - 1, -tot / float(e_real), tot)

    return pl.pallas_call(
        body,
        grid=(nsteps,),
        in_specs=[
            pl.BlockSpec((blk, 128), lambda i: (i, 0)),
            pl.BlockSpec((blk, 128), lambda i: (i, 0)),
        ],
        out_specs=pl.BlockSpec((1, 1), lambda i: (0, 0),
                               memory_space=pltpu.SMEM),
        out_shape=jax.ShapeDtypeStruct((1, 1), jnp.float32),
    )(pos2, neg2)


def kernel(node_features, edge_indices, W, b):
    n, _ = node_features.shape
    h_dim = W.shape[1]
    n_pad = ((n + 2047) // 2048) * 2048
    src = edge_indices[0]
    dst = edge_indices[1]
    e_total = src.shape[0]
    dstm1e = jnp.concatenate([dst[-1:], dst, jnp.zeros((7,), dst.dtype)])

    deg2 = _sc_degree(dst, n_pad)
    g = _tc_encode(node_features, W, deg2)
    acc2 = _sc_scatter(g, src, dst, n_pad)
    z = _tc_normalize(acc2, g, deg2, b.reshape(1, h_dim))
    pos, neg = _sc_dots(z, src, dstm1e)
    loss = _tc_loss(pos.reshape(-1, 128), neg.reshape(-1, 128), e_total)
    return z, loss[0, 0]
